# baseline (speedup 1.0000x reference)
import functools

import jax
import jax.numpy as jnp
from jax import lax
from jax.experimental import pallas as pl
from jax.experimental.pallas import tpu as pltpu
from jax.experimental.pallas import tpu_sc as plsc

_info = plsc.get_sparse_core_info()
_NC, _NS, _L = _info.num_cores, _info.num_subcores, _info.num_lanes
_NW = _NC * _NS

_D = 128
_CHUNK = 64


def _build(b, s, d):
    assert s == _NW * _CHUNK and d == _D
    mesh = plsc.VectorSubcoreMesh(core_axis_name="c", subcore_axis_name="s")

    @functools.partial(
        pl.kernel,
        mesh=mesh,
        out_type=jax.ShapeDtypeStruct((b, _NW, _CHUNK, d), jnp.float32),
        scratch_types=[
            pltpu.VMEM((b, _CHUNK), jnp.int32),
            pltpu.VMEM((b, _CHUNK, d), jnp.float32),
            pltpu.VMEM((_CHUNK, d), jnp.float32),
            pltpu.SemaphoreType.DMA,
            pltpu.SemaphoreType.DMA,
            pltpu.SemaphoreType.DMA,
        ],
    )
    def k(x_hbm, table_hbm, pos_hbm, out_hbm, idx_v, rows_v, pos_v,
          sem_p, sem_g, sem_w):
        wid = lax.axis_index("s") * _NC + lax.axis_index("c")
        pltpu.sync_copy(x_hbm.at[wid], idx_v)
        g_cps = [
            pltpu.async_copy(table_hbm.at[idx_v.at[j]], rows_v.at[j], sem_g)
            for j in range(b)
        ]
        pos_cp = pltpu.async_copy(pos_hbm.at[wid], pos_v, sem_p)
        pos_cp.wait()

        w_cps = []
        for j in range(b):
            g_cps[j].wait()

            def body(r, carry, j=j):
                for c in range(d // _L):
                    sl = pl.ds(c * _L, _L)
                    rows_v[j, r, sl] = rows_v[j, r, sl] + pos_v[r, sl]
                return carry

            lax.fori_loop(0, _CHUNK, body, 0, unroll=4)
            w_cps.append(
                pltpu.async_copy(rows_v.at[j], out_hbm.at[j, wid], sem_w)
            )
        for cp in w_cps:
            cp.wait()

    return k


def kernel(X, token_table, pos_embed):
    b, s = X.shape
    vocab, d = token_table.shape
    xt = X.reshape(b, _NW, _CHUNK).transpose(1, 0, 2).astype(jnp.int32)
    pos3 = pos_embed.reshape(_NW, _CHUNK, d).astype(jnp.float32)
    out = _build(b, s, d)(xt, token_table, pos3)
    return out.reshape(b, s, d)


# speedup vs baseline: 1.1866x; 1.1866x over previous
import functools

import jax
import jax.numpy as jnp
from jax import lax
from jax.experimental import pallas as pl
from jax.experimental.pallas import tpu as pltpu
from jax.experimental.pallas import tpu_sc as plsc

_info = plsc.get_sparse_core_info()
_NC, _NS, _L = _info.num_cores, _info.num_subcores, _info.num_lanes
_NW = _NC * _NS

_CHUNK = 64
_CPW = 4


def _build(b, s, max_len, d):
    rows_per_w = _CPW * _CHUNK
    assert b * s == _NW * rows_per_w and s % rows_per_w == 0
    w_per_b = s // rows_per_w
    mesh = plsc.VectorSubcoreMesh(core_axis_name="c", subcore_axis_name="s")

    @functools.partial(
        pl.kernel,
        mesh=mesh,
        out_type=jax.ShapeDtypeStruct((b, s, d), jnp.float32),
        scratch_types=[
            pltpu.VMEM((rows_per_w,), jnp.int32),
            pltpu.VMEM((_CPW, _CHUNK, d), jnp.float32),
            pltpu.SemaphoreType.DMA,
            pltpu.SemaphoreType.DMA,
            pltpu.SemaphoreType.DMA,
        ],
    )
    def k(x_hbm, table_hbm, pos_hbm, out_hbm, idx_v, rows_v, sem_p, sem_g,
          sem_w):
        wid = lax.axis_index("s") * _NC + lax.axis_index("c")
        b_idx = wid // w_per_b
        s0 = (wid % w_per_b) * rows_per_w
        pltpu.sync_copy(x_hbm.at[b_idx, pl.ds(s0, rows_per_w)], idx_v)
        pos_cps = [
            pltpu.async_copy(
                pos_hbm.at[0, pl.ds(s0 + j * _CHUNK, _CHUNK)],
                rows_v.at[j],
                sem_p,
            )
            for j in range(_CPW)
        ]
        g_cps = []
        for j in range(_CPW):
            pos_cps[j].wait()
            g_cps.append(
                pltpu.async_copy(
                    table_hbm.at[idx_v.at[pl.ds(j * _CHUNK, _CHUNK)]],
                    rows_v.at[j],
                    sem_g,
                    add=True,
                )
            )
        w_cps = []
        for j in range(_CPW):
            g_cps[j].wait()
            w_cps.append(
                pltpu.async_copy(
                    rows_v.at[j],
                    out_hbm.at[b_idx, pl.ds(s0 + j * _CHUNK, _CHUNK)],
                    sem_w,
                )
            )
        for cp in w_cps:
            cp.wait()

    return k


def kernel(X, token_table, pos_embed):
    b, s = X.shape
    vocab, d = token_table.shape
    _, max_len, _ = pos_embed.shape
    return _build(b, s, max_len, d)(X.astype(jnp.int32), token_table, pos_embed)


# speedup vs baseline: 1.1951x; 1.0072x over previous
import functools

import jax
import jax.numpy as jnp
from jax import lax
from jax.experimental import pallas as pl
from jax.experimental.pallas import tpu as pltpu
from jax.experimental.pallas import tpu_sc as plsc

_info = plsc.get_sparse_core_info()
_NC, _NS, _L = _info.num_cores, _info.num_subcores, _info.num_lanes
_NW = _NC * _NS

_CHUNK = 64


def _build(b, s, d):
    assert s == _NW * _CHUNK
    mesh = plsc.VectorSubcoreMesh(core_axis_name="c", subcore_axis_name="s")

    @functools.partial(
        pl.kernel,
        mesh=mesh,
        out_type=jax.ShapeDtypeStruct((b, s, d), jnp.float32),
        scratch_types=[
            pltpu.VMEM((b, _CHUNK), jnp.int32),
            pltpu.VMEM((b, _CHUNK, d), jnp.float32),
            pltpu.VMEM_SHARED((_NS, _CHUNK, d), jnp.float32),
            pltpu.SemaphoreType.DMA,
            pltpu.SemaphoreType.DMA,
            pltpu.SemaphoreType.DMA,
        ],
    )
    def k(x_hbm, table_hbm, pos_hbm, out_hbm, idx_v, rows_v, pos_sh,
          sem_p, sem_g, sem_w):
        sid = lax.axis_index("s")
        wid = sid * _NC + lax.axis_index("c")
        s0 = wid * _CHUNK
        i_cps = [
            pltpu.async_copy(
                x_hbm.at[j, pl.ds(s0, _CHUNK)], idx_v.at[j], sem_p
            )
            for j in range(b)
        ]
        pos_cp = pltpu.async_copy(pos_hbm.at[0, pl.ds(s0, _CHUNK)],
                                  pos_sh.at[sid], sem_p)
        pos_cp.wait()
        r_cps = [
            pltpu.async_copy(pos_sh.at[sid], rows_v.at[j], sem_p)
            for j in range(b)
        ]
        for cp in i_cps:
            cp.wait()
        g_cps = []
        for j in range(b):
            r_cps[j].wait()
            g_cps.append(
                pltpu.async_copy(
                    table_hbm.at[idx_v.at[j]], rows_v.at[j], sem_g, add=True
                )
            )
        w_cps = []
        for j in range(b):
            g_cps[j].wait()
            w_cps.append(
                pltpu.async_copy(
                    rows_v.at[j], out_hbm.at[j, pl.ds(s0, _CHUNK)], sem_w
                )
            )
        for cp in w_cps:
            cp.wait()

    return k


def kernel(X, token_table, pos_embed):
    b, s = X.shape
    vocab, d = token_table.shape
    return _build(b, s, d)(X.astype(jnp.int32), token_table, pos_embed)
